# Initial kernel scaffold; baseline (speedup 1.0000x reference)
import jax
import jax.numpy as jnp
from jax.experimental import pallas as pl


def kernel(sentence, sentence_len, embedding_weight):
    raise NotImplementedError("write your pallas kernel here")


# speedup vs baseline: 11.4506x; 11.4506x over previous
import dataclasses
import functools

import jax
import jax.numpy as jnp
from jax import lax
from jax.experimental import pallas as pl
from jax.experimental.pallas import tpu as pltpu
from jax.experimental.pallas import tpu_sc as plsc

SEQ = 200
BATCH = 4096
D = 128
NC = 2
NS = 16
NW = NC * NS
BW = BATCH // NW


def _body(sent_ref, len_ref, tab_ref, out_ref,
          idx_v, rows_a, rows_b, obuf, scat_idx, len_v, recip_v,
          acc_sh, sem_a, sem_b):
    cid = lax.axis_index("c")
    sid = lax.axis_index("s")
    wid = cid * NS + sid
    base = wid * BW

    pltpu.sync_copy(sent_ref.at[:, pl.ds(base, BW)], idx_v)
    pltpu.sync_copy(len_ref.at[pl.ds(base, BW)], len_v)

    @pl.loop(0, BW, step=16)
    def _(j):
        scat_idx[pl.ds(j, 16)] = lax.iota(jnp.int32, 16) + (j + sid * BW)
        recip_v[pl.ds(j, 16)] = 1.0 / len_v[pl.ds(j, 16)].astype(jnp.float32)

    pltpu.async_copy(tab_ref.at[idx_v.at[0]], rows_a, sem_a)

    @pl.loop(0, SEQ, step=2)
    def _(t):
        pltpu.async_copy(tab_ref.at[idx_v.at[t + 1]], rows_b, sem_b)
        pltpu.make_async_copy(tab_ref.at[idx_v.at[t]], rows_a, sem_a).wait()

        @pl.when(t == 0)
        def _():
            pltpu.sync_copy(rows_a, acc_sh.at[scat_idx])

        @pl.when(t > 0)
        def _():
            pltpu.sync_copy(rows_a, acc_sh.at[scat_idx], add=True)

        @pl.when(t + 2 < SEQ)
        def _():
            pltpu.async_copy(tab_ref.at[idx_v.at[t + 2]], rows_a, sem_a)

        pltpu.make_async_copy(tab_ref.at[idx_v.at[t + 1]], rows_b, sem_b).wait()
        pltpu.sync_copy(rows_b, acc_sh.at[scat_idx], add=True)

    pltpu.sync_copy(acc_sh.at[pl.ds(sid * BW, BW)], obuf)

    @pl.loop(0, BW)
    def _(r):
        rec = plsc.load_gather(recip_v, [jnp.full((16,), r, dtype=jnp.int32)])

        @pl.loop(0, D, step=16)
        def _(c):
            obuf[r, pl.ds(c, 16)] = obuf[r, pl.ds(c, 16)] * rec

    pltpu.sync_copy(obuf, out_ref.at[pl.ds(base, BW)])


def kernel(sentence, sentence_len, embedding_weight):
    mesh = plsc.VectorSubcoreMesh(core_axis_name="c", subcore_axis_name="s")
    cp = pltpu.CompilerParams()
    if "needs_layout_passes" in pltpu.CompilerParams.__dataclass_fields__:
        cp = dataclasses.replace(cp, needs_layout_passes=False)
    run = functools.partial(
        pl.kernel,
        compiler_params=cp,
        out_type=jax.ShapeDtypeStruct((BATCH, D), jnp.float32),
        mesh=mesh,
        scratch_types=[
            pltpu.VMEM((SEQ, BW), jnp.int32),
            pltpu.VMEM((BW, D), jnp.float32),
            pltpu.VMEM((BW, D), jnp.float32),
            pltpu.VMEM((BW, D), jnp.float32),
            pltpu.VMEM((BW,), jnp.int32),
            pltpu.VMEM((BW,), jnp.int32),
            pltpu.VMEM((BW,), jnp.float32),
            pltpu.VMEM_SHARED((NS * BW, D), jnp.float32),
            pltpu.SemaphoreType.DMA,
            pltpu.SemaphoreType.DMA,
        ],
    )(_body)
    return run(sentence, sentence_len, embedding_weight)
